# baseline (speedup 1.0000x reference)
import functools

import jax
import jax.numpy as jnp
from jax import lax
from jax.experimental import pallas as pl
from jax.experimental.pallas import tpu as pltpu
from jax.experimental.pallas import tpu_sc as plsc

B, NPG, D = 10, 1000, 128
N = B * NPG
NP = 10240
E = 320000
NC, NS = 2, 16
NW = NC * NS
EPW = E // NW
CH, K = 125, 80
ROWS_PT = NP // NS

_mesh = plsc.VectorSubcoreMesh(core_axis_name="c", subcore_axis_name="s")


@functools.partial(
    pl.kernel,
    out_type=jax.ShapeDtypeStruct((NC, NP), jnp.float32),
    mesh=_mesh,
    scratch_types=[
        pltpu.VMEM((K, CH), jnp.int32),
        pltpu.VMEM((128,), jnp.float32),
        pltpu.VMEM((NP,), jnp.float32),
        pltpu.VMEM_SHARED((NP,), jnp.float32),
    ],
)
def _deg_kernel(dst_hbm, out_hbm, idx_v, ones_v, zero_v, acc_s):
    c = lax.axis_index("c")
    s = lax.axis_index("s")
    pltpu.sync_copy(dst_hbm.at[c, s], idx_v)
    one16 = jnp.ones((16,), jnp.float32)
    for i in range(8):
        ones_v[pl.ds(i * 16, 16)] = one16

    @pl.when(s == 0)
    def _():
        zero16 = jnp.zeros((16,), jnp.float32)

        def zbody(i, carry):
            zero_v[pl.ds(i * 16, 16)] = zero16
            return carry

        lax.fori_loop(0, NP // 16, zbody, 0)
        pltpu.sync_copy(zero_v, acc_s)

    plsc.subcore_barrier()

    def body(j, carry):
        pltpu.sync_copy(ones_v.at[pl.ds(0, CH)], acc_s.at[idx_v.at[j]], add=True)
        return carry

    lax.fori_loop(0, K, body, 0)
    plsc.subcore_barrier()
    base = s * ROWS_PT
    pltpu.sync_copy(acc_s.at[pl.ds(base, ROWS_PT)],
                    out_hbm.at[c].at[pl.ds(base, ROWS_PT)])


@functools.partial(
    pl.kernel,
    out_type=jax.ShapeDtypeStruct((NC, NP, D), jnp.float32),
    mesh=_mesh,
    scratch_types=[
        pltpu.VMEM((EPW,), jnp.int32),
        pltpu.VMEM((CH, K), jnp.int32),
        pltpu.VMEM((K, D), jnp.float32),
        pltpu.VMEM((K, D), jnp.float32),
        pltpu.VMEM_SHARED((NP, D), jnp.float32),
        pltpu.SemaphoreType.DMA,
        pltpu.SemaphoreType.DMA,
        pltpu.SemaphoreType.DMA,
        pltpu.SemaphoreType.DMA,
    ],
)
def _mp_kernel(hw_hbm, src_hbm, dst_hbm, out_hbm,
               src_v, dst_v, buf0, buf1, acc_s, sem0, sem0b, sem1, sem1b):
    c = lax.axis_index("c")
    s = lax.axis_index("s")
    pltpu.sync_copy(src_hbm.at[c, s], src_v)
    pltpu.sync_copy(dst_hbm.at[c, s], dst_v)

    H = K // 2

    def _gather(j, buf, sa, sb):
        pltpu.async_copy(hw_hbm.at[src_v.at[pl.ds(j * K, H)]],
                         buf.at[pl.ds(0, H)], sa)
        pltpu.async_copy(hw_hbm.at[src_v.at[pl.ds(j * K + H, H)]],
                         buf.at[pl.ds(H, H)], sb)

    def _wait(buf, sa, sb):
        pltpu.make_async_copy(hw_hbm.at[pl.ds(0, H)], buf.at[pl.ds(0, H)], sa).wait()
        pltpu.make_async_copy(hw_hbm.at[pl.ds(0, H)], buf.at[pl.ds(H, H)], sb).wait()

    _gather(0, buf0, sem0, sem0b)

    zero16 = jnp.zeros((16,), jnp.float32)

    def zbody(i, carry):
        for t in range(D // 16):
            buf1[i, pl.ds(t * 16, 16)] = zero16
        return carry

    lax.fori_loop(0, K, zbody, 0)
    base = s * ROWS_PT
    for r in range(ROWS_PT // K):
        pltpu.sync_copy(buf1, acc_s.at[pl.ds(base + r * K, K)])
    plsc.subcore_barrier()

    def body(i, carry):
        j0 = 2 * i
        _wait(buf0, sem0, sem0b)
        _gather(j0 + 1, buf1, sem1, sem1b)
        pltpu.sync_copy(buf0, acc_s.at[dst_v.at[j0]], add=True)
        _wait(buf1, sem1, sem1b)
        _gather(j0 + 2, buf0, sem0, sem0b)
        pltpu.sync_copy(buf1, acc_s.at[dst_v.at[j0 + 1]], add=True)
        return carry

    lax.fori_loop(0, CH // 2, body, 0)
    _wait(buf0, sem0, sem0b)
    pltpu.sync_copy(buf0, acc_s.at[dst_v.at[CH - 1]], add=True)
    plsc.subcore_barrier()
    pltpu.sync_copy(acc_s.at[pl.ds(base, ROWS_PT)],
                    out_hbm.at[c].at[pl.ds(base, ROWS_PT)])


RB = 2000
RB3 = 2000
_DN = (((1,), (1,)), ((), ()))


def _tc1a_body(x_ref, w_ref, hw_ref):
    hw_ref[...] = lax.dot_general(
        x_ref[...], w_ref[...], _DN, preferred_element_type=jnp.float32)


_tc1a = pl.pallas_call(
    _tc1a_body,
    grid=(N // RB,),
    in_specs=[
        pl.BlockSpec((RB, D), lambda i: (i, 0)),
        pl.BlockSpec((D, D), lambda i: (0, 0)),
    ],
    out_specs=pl.BlockSpec((RB, D), lambda i: (i, 0)),
    out_shape=jax.ShapeDtypeStruct((N, D), jnp.float32),
)


def _tc1b_body(hw_ref, deg_ref, hwp_ref, dis_ref):
    deg = deg_ref[0, :, 0] + deg_ref[1, :, 0] + 1.0
    dis = lax.rsqrt(deg)[:, None]
    dis_ref[...] = dis
    hwp_ref[...] = hw_ref[...] * dis


_tc1b = pl.pallas_call(
    _tc1b_body,
    grid=(N // RB,),
    in_specs=[
        pl.BlockSpec((RB, D), lambda i: (i, 0)),
        pl.BlockSpec((NC, RB, 1), lambda i: (0, i, 0)),
    ],
    out_specs=[
        pl.BlockSpec((RB, D), lambda i: (i, 0)),
        pl.BlockSpec((RB, 1), lambda i: (i, 0)),
    ],
    out_shape=[
        jax.ShapeDtypeStruct((N, D), jnp.float32),
        jax.ShapeDtypeStruct((N, 1), jnp.float32),
    ],
)


def _tc2_body(s_ref, hw_ref, dis_ref, b_ref, w_ref, out_ref):
    t = s_ref[0] + s_ref[1] + hw_ref[...]
    h = jnp.maximum(dis_ref[...] * t + b_ref[...], 0.0)
    out_ref[...] = lax.dot_general(
        h, w_ref[...], _DN, preferred_element_type=jnp.float32) * dis_ref[...]


_tc2 = pl.pallas_call(
    _tc2_body,
    grid=(N // RB,),
    in_specs=[
        pl.BlockSpec((NC, RB, D), lambda i: (0, i, 0)),
        pl.BlockSpec((RB, D), lambda i: (i, 0)),
        pl.BlockSpec((RB, 1), lambda i: (i, 0)),
        pl.BlockSpec((1, D), lambda i: (0, 0)),
        pl.BlockSpec((D, D), lambda i: (0, 0)),
    ],
    out_specs=pl.BlockSpec((RB, D), lambda i: (i, 0)),
    out_shape=jax.ShapeDtypeStruct((N, D), jnp.float32),
)


def _tc3_body(s_ref, hw_ref, dis_ref, b_ref, x_ref, out_ref):
    t = s_ref[0] + s_ref[1] + hw_ref[...]
    h = jnp.maximum(dis_ref[...] * t + b_ref[...], 0.0)
    out_ref[...] = x_ref[...] + h


_tc3 = pl.pallas_call(
    _tc3_body,
    grid=(N // RB3,),
    in_specs=[
        pl.BlockSpec((NC, RB3, D), lambda i: (0, i, 0)),
        pl.BlockSpec((RB3, D), lambda i: (i, 0)),
        pl.BlockSpec((RB3, 1), lambda i: (i, 0)),
        pl.BlockSpec((1, D), lambda i: (0, 0)),
        pl.BlockSpec((RB3, D), lambda i: (i, 0)),
    ],
    out_specs=pl.BlockSpec((RB3, D), lambda i: (i, 0)),
    out_shape=jax.ShapeDtypeStruct((N, D), jnp.float32),
)


def kernel(x, edge_index, x_mask, W1, b1, W2, b2):
    del x_mask
    x2d = x.reshape(N, D)
    src = edge_index[0].reshape(NC, NS, EPW)
    dst = edge_index[1].reshape(NC, NS, CH, K)
    b1r = b1.reshape(1, D)
    b2r = b2.reshape(1, D)

    degp = _deg_kernel(dst.reshape(NC, NS, K, CH))
    hw1raw = _tc1a(x2d, W1)
    hw1p, dis = _tc1b(hw1raw, degp.reshape(NC, NP, 1))
    s1 = _mp_kernel(hw1p, src, dst)
    hw2p = _tc2(s1, hw1p, dis, b1r, W2)
    s2 = _mp_kernel(hw2p, src, dst)
    out2d = _tc3(s2, hw2p, dis, b2r, x2d)
    return out2d.reshape(B, NPG, D)


# speedup vs baseline: 1.0137x; 1.0137x over previous
import functools

import jax
import jax.numpy as jnp
from jax import lax
from jax.experimental import pallas as pl
from jax.experimental.pallas import tpu as pltpu
from jax.experimental.pallas import tpu_sc as plsc

B, NPG, D = 10, 1000, 128
N = B * NPG
NP = 10240
E = 320000
NC, NS = 2, 16
NW = NC * NS
EPW = E // NW
CH, K = 125, 80
ROWS_PT = NP // NS

_mesh = plsc.VectorSubcoreMesh(core_axis_name="c", subcore_axis_name="s")


@functools.partial(
    pl.kernel,
    out_type=jax.ShapeDtypeStruct((NC, NP), jnp.float32),
    mesh=_mesh,
    scratch_types=[
        pltpu.VMEM((K, CH), jnp.int32),
        pltpu.VMEM((128,), jnp.float32),
        pltpu.VMEM((NP,), jnp.float32),
        pltpu.VMEM_SHARED((NP,), jnp.float32),
    ],
)
def _deg_kernel(dst_hbm, out_hbm, idx_v, ones_v, zero_v, acc_s):
    c = lax.axis_index("c")
    s = lax.axis_index("s")
    pltpu.sync_copy(dst_hbm.at[c, s], idx_v)
    one16 = jnp.ones((16,), jnp.float32)
    for i in range(8):
        ones_v[pl.ds(i * 16, 16)] = one16

    @pl.when(s == 0)
    def _():
        zero16 = jnp.zeros((16,), jnp.float32)

        def zbody(i, carry):
            zero_v[pl.ds(i * 16, 16)] = zero16
            return carry

        lax.fori_loop(0, NP // 16, zbody, 0)
        pltpu.sync_copy(zero_v, acc_s)

    plsc.subcore_barrier()

    def body(j, carry):
        pltpu.sync_copy(ones_v.at[pl.ds(0, CH)], acc_s.at[idx_v.at[j]], add=True)
        return carry

    lax.fori_loop(0, K, body, 0)
    plsc.subcore_barrier()
    base = s * ROWS_PT
    pltpu.sync_copy(acc_s.at[pl.ds(base, ROWS_PT)],
                    out_hbm.at[c].at[pl.ds(base, ROWS_PT)])


@functools.partial(
    pl.kernel,
    out_type=jax.ShapeDtypeStruct((NC, NP, D), jnp.float32),
    mesh=_mesh,
    scratch_types=[
        pltpu.VMEM((EPW,), jnp.int32),
        pltpu.VMEM((CH, K), jnp.int32),
        pltpu.VMEM((K, D), jnp.float32),
        pltpu.VMEM((K, D), jnp.float32),
        pltpu.VMEM_SHARED((NP, D), jnp.float32),
        pltpu.SemaphoreType.DMA,
        pltpu.SemaphoreType.DMA,
        pltpu.SemaphoreType.DMA,
        pltpu.SemaphoreType.DMA,
    ],
)
def _mp_kernel(hw_hbm, src_hbm, dst_hbm, out_hbm,
               src_v, dst_v, buf0, buf1, acc_s, sem0, sem0b, sem1, sem1b):
    c = lax.axis_index("c")
    s = lax.axis_index("s")
    pltpu.sync_copy(src_hbm.at[c, s], src_v)
    pltpu.sync_copy(dst_hbm.at[c, s], dst_v)

    H = K // 2

    def _gather(j, buf, sa, sb):
        pltpu.async_copy(hw_hbm.at[src_v.at[pl.ds(j * K, H)]],
                         buf.at[pl.ds(0, H)], sa)
        pltpu.async_copy(hw_hbm.at[src_v.at[pl.ds(j * K + H, H)]],
                         buf.at[pl.ds(H, H)], sb)

    def _wait(buf, sa, sb):
        pltpu.make_async_copy(hw_hbm.at[pl.ds(0, H)], buf.at[pl.ds(0, H)], sa).wait()
        pltpu.make_async_copy(hw_hbm.at[pl.ds(0, H)], buf.at[pl.ds(H, H)], sb).wait()

    _gather(0, buf0, sem0, sem0b)

    zero16 = jnp.zeros((16,), jnp.float32)

    def zbody(i, carry):
        for t in range(D // 16):
            buf1[i, pl.ds(t * 16, 16)] = zero16
        return carry

    lax.fori_loop(0, K, zbody, 0)
    base = s * ROWS_PT
    for r in range(ROWS_PT // K):
        pltpu.sync_copy(buf1, acc_s.at[pl.ds(base + r * K, K)])
    plsc.subcore_barrier()

    def body(i, carry):
        j0 = 2 * i
        _wait(buf0, sem0, sem0b)
        _gather(j0 + 1, buf1, sem1, sem1b)
        pltpu.sync_copy(buf0, acc_s.at[dst_v.at[j0]], add=True)
        _wait(buf1, sem1, sem1b)
        _gather(j0 + 2, buf0, sem0, sem0b)
        pltpu.sync_copy(buf1, acc_s.at[dst_v.at[j0 + 1]], add=True)
        return carry

    lax.fori_loop(0, CH // 2, body, 0)
    _wait(buf0, sem0, sem0b)
    pltpu.sync_copy(buf0, acc_s.at[dst_v.at[CH - 1]], add=True)
    plsc.subcore_barrier()
    pltpu.sync_copy(acc_s.at[pl.ds(base, ROWS_PT)],
                    out_hbm.at[c].at[pl.ds(base, ROWS_PT)])


RB = 2048
RB3 = 2000
_DN = (((1,), (1,)), ((), ()))


def _tc1a_body(x_ref, w_ref, hw_ref):
    hw_ref[...] = lax.dot_general(
        x_ref[...], w_ref[...], _DN, preferred_element_type=jnp.float32)


_tc1a = pl.pallas_call(
    _tc1a_body,
    grid=(NP // RB,),
    in_specs=[
        pl.BlockSpec((RB, D), lambda i: (i, 0)),
        pl.BlockSpec((D, D), lambda i: (0, 0)),
    ],
    out_specs=pl.BlockSpec((RB, D), lambda i: (i, 0)),
    out_shape=jax.ShapeDtypeStruct((NP, D), jnp.float32),
)


def _tc1b_body(hw_ref, deg_ref, hwp_ref, dis_ref):
    deg = deg_ref[0, :] + deg_ref[1, :] + 1.0
    dis = lax.rsqrt(deg)[:, None]
    dis_ref[...] = dis
    hwp_ref[...] = hw_ref[...] * dis


_tc1b = pl.pallas_call(
    _tc1b_body,
    grid=(NP // RB,),
    in_specs=[
        pl.BlockSpec((RB, D), lambda i: (i, 0)),
        pl.BlockSpec((NC, RB), lambda i: (0, i)),
    ],
    out_specs=[
        pl.BlockSpec((RB, D), lambda i: (i, 0)),
        pl.BlockSpec((RB, 1), lambda i: (i, 0)),
    ],
    out_shape=[
        jax.ShapeDtypeStruct((NP, D), jnp.float32),
        jax.ShapeDtypeStruct((NP, 1), jnp.float32),
    ],
)


def _tc2_body(s_ref, hw_ref, dis_ref, b_ref, w_ref, out_ref):
    t = s_ref[0] + s_ref[1] + hw_ref[...]
    h = jnp.maximum(dis_ref[...] * t + b_ref[...], 0.0)
    out_ref[...] = lax.dot_general(
        h, w_ref[...], _DN, preferred_element_type=jnp.float32) * dis_ref[...]


_tc2 = pl.pallas_call(
    _tc2_body,
    grid=(NP // RB,),
    in_specs=[
        pl.BlockSpec((NC, RB, D), lambda i: (0, i, 0)),
        pl.BlockSpec((RB, D), lambda i: (i, 0)),
        pl.BlockSpec((RB, 1), lambda i: (i, 0)),
        pl.BlockSpec((1, D), lambda i: (0, 0)),
        pl.BlockSpec((D, D), lambda i: (0, 0)),
    ],
    out_specs=pl.BlockSpec((RB, D), lambda i: (i, 0)),
    out_shape=jax.ShapeDtypeStruct((NP, D), jnp.float32),
)


def _tc3_body(s_ref, hw_ref, dis_ref, b_ref, x_ref, out_ref):
    t = s_ref[0] + s_ref[1] + hw_ref[...]
    h = jnp.maximum(dis_ref[...] * t + b_ref[...], 0.0)
    out_ref[...] = x_ref[...] + h


_tc3 = pl.pallas_call(
    _tc3_body,
    grid=(N // RB3,),
    in_specs=[
        pl.BlockSpec((NC, RB3, D), lambda i: (0, i, 0)),
        pl.BlockSpec((RB3, D), lambda i: (i, 0)),
        pl.BlockSpec((RB3, 1), lambda i: (i, 0)),
        pl.BlockSpec((1, D), lambda i: (0, 0)),
        pl.BlockSpec((RB3, D), lambda i: (i, 0)),
    ],
    out_specs=pl.BlockSpec((RB3, D), lambda i: (i, 0)),
    out_shape=jax.ShapeDtypeStruct((N, D), jnp.float32),
)


def kernel(x, edge_index, x_mask, W1, b1, W2, b2):
    del x_mask
    x2d = x.reshape(N, D)
    xpad = jnp.pad(x2d, ((0, NP - N), (0, 0)))
    src = edge_index[0].reshape(NC, NS, EPW)
    dst = edge_index[1].reshape(NC, NS, CH, K)
    b1r = b1.reshape(1, D)
    b2r = b2.reshape(1, D)

    degp = _deg_kernel(dst.reshape(NC, NS, K, CH))
    hw1raw = _tc1a(xpad, W1)
    hw1p, dis = _tc1b(hw1raw, degp)
    s1 = _mp_kernel(hw1p, src, dst)
    hw2p = _tc2(s1, hw1p, dis, b1r, W2)
    s2 = _mp_kernel(hw2p, src, dst)
    out2d = _tc3(s2, hw2p, dis, b2r, x2d)
    return out2d.reshape(B, NPG, D)
